# baseline (speedup 1.0000x reference)
import jax
import jax.numpy as jnp
from jax import lax
from jax.experimental import pallas as pl
from jax.experimental.pallas import tpu as pltpu
from jax.experimental.pallas import tpu_sc as plsc

N_NODES = 10000
N_EDGES = 160000
D_FEAT = 256
D_HALF = 128

NS = 16
E_PAD = 163840
E_PER = E_PAD // NS
ROWS_PER = N_NODES // NS
BATCH = 128
DUMMY_ROW = N_NODES
DEG_ROWS = 10240


def _make_segsum(want_deg):
    def body(*refs):
        if want_deg:
            (x01_hbm, src_hbm, dst_hbm, s_hbm, deg_hbm,
             idx_v, dst_v, rows_v, ones_v, acc_sh, deg_sh) = refs
        else:
            (x01_hbm, src_hbm, dst_hbm, s_hbm,
             idx_v, dst_v, rows_v, acc_sh) = refs

        c = lax.axis_index("c")
        s = lax.axis_index("s")
        ebase = s * E_PER
        rbase = s * ROWS_PER

        pltpu.sync_copy(src_hbm.at[pl.ds(ebase, E_PER)], idx_v)
        pltpu.sync_copy(dst_hbm.at[pl.ds(ebase, E_PER)], dst_v)

        coff = c * N_NODES

        @pl.loop(0, E_PER, step=16)
        def _(i):
            idx_v[pl.ds(i, 16)] = idx_v[pl.ds(i, 16)] + coff

        zero16 = jnp.zeros((16,), jnp.float32)

        @pl.loop(0, BATCH)
        def _(r):
            @pl.loop(0, D_HALF, step=16)
            def _(k):
                rows_v[r, pl.ds(k, 16)] = zero16
            if want_deg:
                ones_v[r, :] = zero16

        @pl.loop(0, 5)
        def _(j):
            pltpu.sync_copy(rows_v.at[pl.ds(0, 125)],
                            acc_sh.at[pl.ds(rbase + j * 125, 125)])

        @pl.when(s == NS - 1)
        def _():
            pltpu.sync_copy(rows_v.at[pl.ds(0, 1)],
                            acc_sh.at[pl.ds(N_NODES, 1)])

        if want_deg:
            dbase = s * (DEG_ROWS // NS)

            @pl.loop(0, 5)
            def _(j):
                pltpu.sync_copy(ones_v, deg_sh.at[pl.ds(dbase + j * BATCH,
                                                        BATCH)])

            e0 = jnp.where(lax.iota(jnp.int32, 16) == 0, 1.0, 0.0)

            @pl.loop(0, BATCH)
            def _(r):
                ones_v[r, :] = e0

        plsc.subcore_barrier()

        @pl.loop(0, E_PER // BATCH)
        def _(b):
            off = b * BATCH
            idx_sl = idx_v.at[pl.ds(off, BATCH)]
            dst_sl = dst_v.at[pl.ds(off, BATCH)]
            pltpu.sync_copy(x01_hbm.at[idx_sl], rows_v)
            pltpu.sync_copy(rows_v, acc_sh.at[dst_sl], add=True)
            if want_deg:
                pltpu.sync_copy(ones_v, deg_sh.at[dst_sl], add=True)

        plsc.subcore_barrier()

        @pl.when(c == 0)
        def _():
            pltpu.sync_copy(acc_sh.at[pl.ds(rbase, ROWS_PER)],
                            s_hbm.at[0, pl.ds(rbase, ROWS_PER)])
            if want_deg:
                pltpu.sync_copy(deg_sh.at[pl.ds(rbase, ROWS_PER)],
                                deg_hbm.at[pl.ds(rbase, ROWS_PER)])

        @pl.when(c == 1)
        def _():
            pltpu.sync_copy(acc_sh.at[pl.ds(rbase, ROWS_PER)],
                            s_hbm.at[1, pl.ds(rbase, ROWS_PER)])

    s_type = jax.ShapeDtypeStruct((2, N_NODES, D_HALF), jnp.float32)
    deg_type = jax.ShapeDtypeStruct((N_NODES, 16), jnp.float32)
    scratch = [
        pltpu.VMEM((E_PER,), jnp.int32),
        pltpu.VMEM((E_PER,), jnp.int32),
        pltpu.VMEM((BATCH, D_HALF), jnp.float32),
    ]
    if want_deg:
        scratch += [pltpu.VMEM((BATCH, 16), jnp.float32)]
    scratch += [pltpu.VMEM_SHARED((N_NODES + 1, D_HALF), jnp.float32)]
    if want_deg:
        scratch += [pltpu.VMEM_SHARED((DEG_ROWS, 16), jnp.float32)]

    return pl.kernel(
        body,
        out_type=(s_type, deg_type) if want_deg else s_type,
        mesh=plsc.VectorSubcoreMesh(core_axis_name="c", subcore_axis_name="s"),
        compiler_params=pltpu.CompilerParams(use_tc_tiling_on_sc=False),
        scratch_types=scratch,
    )


_segsum_deg = _make_segsum(True)
_segsum_nodeg = _make_segsum(False)


_R = 2000


def _mm0_body(x_ref, w_ref, y_ref):
    y = jnp.dot(x_ref[...], w_ref[...], preferred_element_type=jnp.float32)
    y_ref[0] = y[:, :D_HALF]
    y_ref[1] = y[:, D_HALF:]


def _mm0(x, w0t):
    return pl.pallas_call(
        _mm0_body,
        grid=(N_NODES // _R,),
        in_specs=[
            pl.BlockSpec((_R, D_FEAT), lambda i: (i, 0)),
            pl.BlockSpec((D_FEAT, D_FEAT), lambda i: (0, 0)),
        ],
        out_specs=pl.BlockSpec((2, _R, D_HALF), lambda i: (0, i, 0)),
        out_shape=jax.ShapeDtypeStruct((2, N_NODES, D_HALF), jnp.float32),
    )(x, w0t)


def _mm1_body(s_ref, deg_ref, w_ref, y_ref):
    dinv = 1.0 / jnp.maximum(deg_ref[:, 0:1], 1.0)
    h0 = jnp.maximum(s_ref[0], 0.0) * dinv
    h1 = jnp.maximum(s_ref[1], 0.0) * dinv
    y = (jnp.dot(h0, w_ref[:D_HALF, :], preferred_element_type=jnp.float32)
         + jnp.dot(h1, w_ref[D_HALF:, :], preferred_element_type=jnp.float32))
    y_ref[0] = y[:, :D_HALF]
    y_ref[1] = y[:, D_HALF:]


def _mm1(s0, deg, w1t):
    return pl.pallas_call(
        _mm1_body,
        grid=(N_NODES // _R,),
        in_specs=[
            pl.BlockSpec((2, _R, D_HALF), lambda i: (0, i, 0)),
            pl.BlockSpec((_R, 16), lambda i: (i, 0)),
            pl.BlockSpec((D_FEAT, D_FEAT), lambda i: (0, 0)),
        ],
        out_specs=pl.BlockSpec((2, _R, D_HALF), lambda i: (0, i, 0)),
        out_shape=jax.ShapeDtypeStruct((2, N_NODES, D_HALF), jnp.float32),
    )(s0, deg, w1t)


def _scale_body(s_ref, deg_ref, o_ref):
    dinv = 1.0 / jnp.maximum(deg_ref[:, 0:1], 1.0)
    o_ref[:, :D_HALF] = s_ref[0] * dinv
    o_ref[:, D_HALF:] = s_ref[1] * dinv


def _scale(s1, deg):
    return pl.pallas_call(
        _scale_body,
        grid=(N_NODES // _R,),
        in_specs=[
            pl.BlockSpec((2, _R, D_HALF), lambda i: (0, i, 0)),
            pl.BlockSpec((_R, 16), lambda i: (i, 0)),
        ],
        out_specs=pl.BlockSpec((_R, D_FEAT), lambda i: (i, 0)),
        out_shape=jax.ShapeDtypeStruct((N_NODES, D_FEAT), jnp.float32),
    )(s1, deg)


def kernel(x, edge_index, W0, W1):
    src = edge_index[0].astype(jnp.int32)
    dst = edge_index[1].astype(jnp.int32)
    pad = E_PAD - N_EDGES
    src = jnp.concatenate([src, jnp.zeros((pad,), jnp.int32)])
    dst = jnp.concatenate([dst, jnp.full((pad,), DUMMY_ROW, jnp.int32)])
    w0t = W0.T
    w1t = W1.T

    y0 = _mm0(x, w0t)
    s0, deg = _segsum_deg(y0.reshape(2 * N_NODES, D_HALF), src, dst)
    y1 = _mm1(s0, deg, w1t)
    s1 = _segsum_nodeg(y1.reshape(2 * N_NODES, D_HALF), src, dst)
    return _scale(s1, deg)


# speedup vs baseline: 1.0038x; 1.0038x over previous
import jax
import jax.numpy as jnp
from jax import lax
from jax.experimental import pallas as pl
from jax.experimental.pallas import tpu as pltpu
from jax.experimental.pallas import tpu_sc as plsc

N_NODES = 10000
N_EDGES = 160000
D_FEAT = 256
D_HALF = 128

NS = 16
E_PAD = 163840
E_PER = E_PAD // NS
ROWS_PER = N_NODES // NS
BATCH = 128
N_DUMMY = 128
DEG_ROWS = 10240


def _make_segsum(want_deg):
    def body(*refs):
        if want_deg:
            (x01_hbm, src_hbm, dst_hbm, s_hbm, deg_hbm,
             idx_v, dst_v, rows_v, ones_v, acc_sh, deg_sh) = refs
        else:
            (x01_hbm, src_hbm, dst_hbm, s_hbm,
             idx_v, dst_v, rows_v, acc_sh) = refs

        c = lax.axis_index("c")
        s = lax.axis_index("s")
        ebase = s * E_PER
        rbase = s * ROWS_PER

        pltpu.sync_copy(src_hbm.at[pl.ds(ebase, E_PER)], idx_v)
        pltpu.sync_copy(dst_hbm.at[pl.ds(ebase, E_PER)], dst_v)

        coff = c * N_NODES

        @pl.loop(0, E_PER, step=16)
        def _(i):
            idx_v[pl.ds(i, 16)] = idx_v[pl.ds(i, 16)] + coff

        zero16 = jnp.zeros((16,), jnp.float32)

        @pl.loop(0, BATCH)
        def _(r):
            @pl.loop(0, D_HALF, step=16)
            def _(k):
                rows_v[r, pl.ds(k, 16)] = zero16
            if want_deg:
                ones_v[r, :] = zero16

        @pl.loop(0, 5)
        def _(j):
            pltpu.sync_copy(rows_v.at[pl.ds(0, 125)],
                            acc_sh.at[pl.ds(rbase + j * 125, 125)])

        @pl.when(s == NS - 1)
        def _():
            pltpu.sync_copy(rows_v.at[pl.ds(0, 1)],
                            acc_sh.at[pl.ds(N_NODES, 1)])

        if want_deg:
            dbase = s * (DEG_ROWS // NS)

            @pl.loop(0, 5)
            def _(j):
                pltpu.sync_copy(ones_v, deg_sh.at[pl.ds(dbase + j * BATCH,
                                                        BATCH)])

            e0 = jnp.where(lax.iota(jnp.int32, 16) == 0, 1.0, 0.0)

            @pl.loop(0, BATCH)
            def _(r):
                ones_v[r, :] = e0

        plsc.subcore_barrier()

        @pl.loop(0, E_PER // BATCH)
        def _(b):
            off = b * BATCH
            idx_sl = idx_v.at[pl.ds(off, BATCH)]
            dst_sl = dst_v.at[pl.ds(off, BATCH)]
            pltpu.sync_copy(x01_hbm.at[idx_sl], rows_v)
            pltpu.sync_copy(rows_v, acc_sh.at[dst_sl], add=True)
            if want_deg:
                pltpu.sync_copy(ones_v, deg_sh.at[dst_sl], add=True)

        plsc.subcore_barrier()

        @pl.when(c == 0)
        def _():
            pltpu.sync_copy(acc_sh.at[pl.ds(rbase, ROWS_PER)],
                            s_hbm.at[0, pl.ds(rbase, ROWS_PER)])
            if want_deg:
                pltpu.sync_copy(deg_sh.at[pl.ds(rbase, ROWS_PER)],
                                deg_hbm.at[pl.ds(rbase, ROWS_PER)])

        @pl.when(c == 1)
        def _():
            pltpu.sync_copy(acc_sh.at[pl.ds(rbase, ROWS_PER)],
                            s_hbm.at[1, pl.ds(rbase, ROWS_PER)])

    s_type = jax.ShapeDtypeStruct((2, N_NODES, D_HALF), jnp.float32)
    deg_type = jax.ShapeDtypeStruct((N_NODES, 16), jnp.float32)
    scratch = [
        pltpu.VMEM((E_PER,), jnp.int32),
        pltpu.VMEM((E_PER,), jnp.int32),
        pltpu.VMEM((BATCH, D_HALF), jnp.float32),
    ]
    if want_deg:
        scratch += [pltpu.VMEM((BATCH, 16), jnp.float32)]
    scratch += [pltpu.VMEM_SHARED((N_NODES + N_DUMMY, D_HALF), jnp.float32)]
    if want_deg:
        scratch += [pltpu.VMEM_SHARED((DEG_ROWS, 16), jnp.float32)]

    return pl.kernel(
        body,
        out_type=(s_type, deg_type) if want_deg else s_type,
        mesh=plsc.VectorSubcoreMesh(core_axis_name="c", subcore_axis_name="s"),
        compiler_params=pltpu.CompilerParams(use_tc_tiling_on_sc=False),
        scratch_types=scratch,
    )


_segsum_deg = _make_segsum(True)
_segsum_nodeg = _make_segsum(False)


_R = 2000


def _mm0_body(x_ref, w_ref, y_ref):
    y = jnp.dot(x_ref[...], w_ref[...], preferred_element_type=jnp.float32)
    y_ref[0] = y[:, :D_HALF]
    y_ref[1] = y[:, D_HALF:]


def _mm0(x, w0t):
    return pl.pallas_call(
        _mm0_body,
        grid=(N_NODES // _R,),
        in_specs=[
            pl.BlockSpec((_R, D_FEAT), lambda i: (i, 0)),
            pl.BlockSpec((D_FEAT, D_FEAT), lambda i: (0, 0)),
        ],
        out_specs=pl.BlockSpec((2, _R, D_HALF), lambda i: (0, i, 0)),
        out_shape=jax.ShapeDtypeStruct((2, N_NODES, D_HALF), jnp.float32),
    )(x, w0t)


def _mm1_body(s_ref, deg_ref, w_ref, y_ref):
    dinv = 1.0 / jnp.maximum(deg_ref[:, 0:1], 1.0)
    h0 = jnp.maximum(s_ref[0], 0.0) * dinv
    h1 = jnp.maximum(s_ref[1], 0.0) * dinv
    y = (jnp.dot(h0, w_ref[:D_HALF, :], preferred_element_type=jnp.float32)
         + jnp.dot(h1, w_ref[D_HALF:, :], preferred_element_type=jnp.float32))
    y_ref[0] = y[:, :D_HALF]
    y_ref[1] = y[:, D_HALF:]


def _mm1(s0, deg, w1t):
    return pl.pallas_call(
        _mm1_body,
        grid=(N_NODES // _R,),
        in_specs=[
            pl.BlockSpec((2, _R, D_HALF), lambda i: (0, i, 0)),
            pl.BlockSpec((_R, 16), lambda i: (i, 0)),
            pl.BlockSpec((D_FEAT, D_FEAT), lambda i: (0, 0)),
        ],
        out_specs=pl.BlockSpec((2, _R, D_HALF), lambda i: (0, i, 0)),
        out_shape=jax.ShapeDtypeStruct((2, N_NODES, D_HALF), jnp.float32),
    )(s0, deg, w1t)


def _scale_body(s_ref, deg_ref, o_ref):
    dinv = 1.0 / jnp.maximum(deg_ref[:, 0:1], 1.0)
    o_ref[:, :D_HALF] = s_ref[0] * dinv
    o_ref[:, D_HALF:] = s_ref[1] * dinv


def _scale(s1, deg):
    return pl.pallas_call(
        _scale_body,
        grid=(N_NODES // _R,),
        in_specs=[
            pl.BlockSpec((2, _R, D_HALF), lambda i: (0, i, 0)),
            pl.BlockSpec((_R, 16), lambda i: (i, 0)),
        ],
        out_specs=pl.BlockSpec((_R, D_FEAT), lambda i: (i, 0)),
        out_shape=jax.ShapeDtypeStruct((N_NODES, D_FEAT), jnp.float32),
    )(s1, deg)


def kernel(x, edge_index, W0, W1):
    src = edge_index[0].astype(jnp.int32)
    dst = edge_index[1].astype(jnp.int32)
    pad = E_PAD - N_EDGES
    src = jnp.concatenate([src, jnp.zeros((pad,), jnp.int32)])
    dst = jnp.concatenate(
        [dst, N_NODES + (jnp.arange(pad, dtype=jnp.int32) % N_DUMMY)])
    w0t = W0.T
    w1t = W1.T

    y0 = _mm0(x, w0t)
    s0, deg = _segsum_deg(y0.reshape(2 * N_NODES, D_HALF), src, dst)
    y1 = _mm1(s0, deg, w1t)
    s1 = _segsum_nodeg(y1.reshape(2 * N_NODES, D_HALF), src, dst)
    return _scale(s1, deg)


# speedup vs baseline: 1.8677x; 1.8607x over previous
import jax
import jax.numpy as jnp
from jax import lax
from jax.experimental import pallas as pl
from jax.experimental.pallas import tpu as pltpu
from jax.experimental.pallas import tpu_sc as plsc

N_NODES = 10000
N_EDGES = 160000
D_FEAT = 256
D_HALF = 128

NS = 16
E_PER = N_EDGES // NS
ROWS_PER = N_NODES // NS
BATCH = 128
NFULL = E_PER // BATCH
TAIL = E_PER - NFULL * BATCH


def _segsum_sc_body(x0_hbm, x1_hbm, src_hbm, dst_hbm, s_hbm, deg_hbm,
                    idx_v, dst_v, rows_v, ones_v, zdeg_v, acc_sh, deg_sh):
    c = lax.axis_index("c")
    s = lax.axis_index("s")
    ebase = s * E_PER
    rbase = s * ROWS_PER

    pltpu.sync_copy(src_hbm.at[pl.ds(ebase, E_PER)], idx_v)
    pltpu.sync_copy(dst_hbm.at[pl.ds(ebase, E_PER)], dst_v)

    zero16 = jnp.zeros((16,), jnp.float32)
    e0 = jnp.where(lax.iota(jnp.int32, 16) == 0, 1.0, 0.0)

    @pl.loop(0, BATCH)
    def _(r):
        @pl.loop(0, D_HALF, step=16)
        def _(k):
            rows_v[r, pl.ds(k, 16)] = zero16
        ones_v[r, :] = e0
        zdeg_v[r, :] = zero16

    @pl.loop(0, 5)
    def _(j):
        pltpu.sync_copy(rows_v.at[pl.ds(0, 125)],
                        acc_sh.at[pl.ds(rbase + j * 125, 125)])

    @pl.when(c == 0)
    def _():
        @pl.loop(0, 5)
        def _(j):
            pltpu.sync_copy(zdeg_v.at[pl.ds(0, 125)],
                            deg_sh.at[pl.ds(rbase + j * 125, 125)])

    plsc.subcore_barrier()

    def do_batch(off, n):
        idx_sl = idx_v.at[pl.ds(off, n)]
        dst_sl = dst_v.at[pl.ds(off, n)]
        rows_sl = rows_v.at[pl.ds(0, n)]

        @pl.when(c == 0)
        def _():
            pltpu.sync_copy(x0_hbm.at[idx_sl], rows_sl)

        @pl.when(c == 1)
        def _():
            pltpu.sync_copy(x1_hbm.at[idx_sl], rows_sl)

        pltpu.sync_copy(rows_sl, acc_sh.at[dst_sl], add=True)

        @pl.when(c == 0)
        def _():
            pltpu.sync_copy(ones_v.at[pl.ds(0, n)], deg_sh.at[dst_sl], add=True)

    @pl.loop(0, NFULL)
    def _(b):
        do_batch(b * BATCH, BATCH)

    do_batch(NFULL * BATCH, TAIL)

    plsc.subcore_barrier()

    @pl.when(c == 0)
    def _():
        pltpu.sync_copy(acc_sh.at[pl.ds(rbase, ROWS_PER)],
                        s_hbm.at[0, pl.ds(rbase, ROWS_PER)])
        pltpu.sync_copy(deg_sh.at[pl.ds(rbase, ROWS_PER)],
                        deg_hbm.at[pl.ds(rbase, ROWS_PER)])

    @pl.when(c == 1)
    def _():
        pltpu.sync_copy(acc_sh.at[pl.ds(rbase, ROWS_PER)],
                        s_hbm.at[1, pl.ds(rbase, ROWS_PER)])


def _segsum(x0, x1, src, dst):
    f = pl.kernel(
        _segsum_sc_body,
        out_type=(
            jax.ShapeDtypeStruct((2, N_NODES, D_HALF), jnp.float32),
            jax.ShapeDtypeStruct((N_NODES, 16), jnp.float32),
        ),
        mesh=plsc.VectorSubcoreMesh(core_axis_name="c", subcore_axis_name="s"),
        compiler_params=pltpu.CompilerParams(use_tc_tiling_on_sc=False),
        scratch_types=[
            pltpu.VMEM((E_PER,), jnp.int32),
            pltpu.VMEM((E_PER,), jnp.int32),
            pltpu.VMEM((BATCH, D_HALF), jnp.float32),
            pltpu.VMEM((BATCH, 16), jnp.float32),
            pltpu.VMEM((BATCH, 16), jnp.float32),
            pltpu.VMEM_SHARED((N_NODES, D_HALF), jnp.float32),
            pltpu.VMEM_SHARED((N_NODES, 16), jnp.float32),
        ],
    )
    return f(x0, x1, src, dst)


_R = 2000


def _mm0_body(x_ref, w_ref, y0_ref, y1_ref):
    y = jnp.dot(x_ref[...], w_ref[...], preferred_element_type=jnp.float32)
    y0_ref[...] = y[:, :D_HALF]
    y1_ref[...] = y[:, D_HALF:]


def _mm0(x, w0t):
    return pl.pallas_call(
        _mm0_body,
        grid=(N_NODES // _R,),
        in_specs=[
            pl.BlockSpec((_R, D_FEAT), lambda i: (i, 0)),
            pl.BlockSpec((D_FEAT, D_FEAT), lambda i: (0, 0)),
        ],
        out_specs=[
            pl.BlockSpec((_R, D_HALF), lambda i: (i, 0)),
            pl.BlockSpec((_R, D_HALF), lambda i: (i, 0)),
        ],
        out_shape=[jax.ShapeDtypeStruct((N_NODES, D_HALF), jnp.float32)] * 2,
    )(x, w0t)


def _mm1_body(s_ref, deg_ref, w_ref, y0_ref, y1_ref):
    dinv = 1.0 / jnp.maximum(deg_ref[:, 0:1], 1.0)
    h0 = jnp.maximum(s_ref[0], 0.0) * dinv
    h1 = jnp.maximum(s_ref[1], 0.0) * dinv
    y = (jnp.dot(h0, w_ref[:D_HALF, :], preferred_element_type=jnp.float32)
         + jnp.dot(h1, w_ref[D_HALF:, :], preferred_element_type=jnp.float32))
    y0_ref[...] = y[:, :D_HALF]
    y1_ref[...] = y[:, D_HALF:]


def _mm1(s0, deg, w1t):
    return pl.pallas_call(
        _mm1_body,
        grid=(N_NODES // _R,),
        in_specs=[
            pl.BlockSpec((2, _R, D_HALF), lambda i: (0, i, 0)),
            pl.BlockSpec((_R, 16), lambda i: (i, 0)),
            pl.BlockSpec((D_FEAT, D_FEAT), lambda i: (0, 0)),
        ],
        out_specs=[
            pl.BlockSpec((_R, D_HALF), lambda i: (i, 0)),
            pl.BlockSpec((_R, D_HALF), lambda i: (i, 0)),
        ],
        out_shape=[jax.ShapeDtypeStruct((N_NODES, D_HALF), jnp.float32)] * 2,
    )(s0, deg, w1t)


def _scale_body(s_ref, deg_ref, o_ref):
    dinv = 1.0 / jnp.maximum(deg_ref[:, 0:1], 1.0)
    o_ref[:, :D_HALF] = s_ref[0] * dinv
    o_ref[:, D_HALF:] = s_ref[1] * dinv


def _scale(s1, deg):
    return pl.pallas_call(
        _scale_body,
        grid=(N_NODES // _R,),
        in_specs=[
            pl.BlockSpec((2, _R, D_HALF), lambda i: (0, i, 0)),
            pl.BlockSpec((_R, 16), lambda i: (i, 0)),
        ],
        out_specs=pl.BlockSpec((_R, D_FEAT), lambda i: (i, 0)),
        out_shape=jax.ShapeDtypeStruct((N_NODES, D_FEAT), jnp.float32),
    )(s1, deg)


def kernel(x, edge_index, W0, W1):
    src = edge_index[0].astype(jnp.int32)
    dst = edge_index[1].astype(jnp.int32)
    w0t = W0.T
    w1t = W1.T

    y00, y01 = _mm0(x, w0t)
    s0, deg = _segsum(y00, y01, src, dst)
    y10, y11 = _mm1(s0, deg, w1t)
    s1, _ = _segsum(y10, y11, src, dst)
    return _scale(s1, deg)


# speedup vs baseline: 1.9144x; 1.0250x over previous
import jax
import jax.numpy as jnp
from jax import lax
from jax.experimental import pallas as pl
from jax.experimental.pallas import tpu as pltpu
from jax.experimental.pallas import tpu_sc as plsc

N_NODES = 10000
N_EDGES = 160000
D_FEAT = 256
D_HALF = 128

NS = 16
E_PER = N_EDGES // NS
ROWS_PER = N_NODES // NS
BATCH = 128
NFULL = E_PER // BATCH
TAIL = E_PER - NFULL * BATCH


def _make_segsum(want_deg):
    def body(*refs):
        if want_deg:
            (x0_hbm, x1_hbm, src_hbm, dst_hbm, s_hbm, deg_hbm,
             idx_v, dst_v, rows_v, ones_v, zdeg_v, acc_sh, deg_sh) = refs
        else:
            (x0_hbm, x1_hbm, src_hbm, dst_hbm, s_hbm,
             idx_v, dst_v, rows_v, acc_sh) = refs

        c = lax.axis_index("c")
        s = lax.axis_index("s")
        ebase = s * E_PER
        rbase = s * ROWS_PER

        pltpu.sync_copy(src_hbm.at[pl.ds(ebase, E_PER)], idx_v)
        pltpu.sync_copy(dst_hbm.at[pl.ds(ebase, E_PER)], dst_v)

        zero16 = jnp.zeros((16,), jnp.float32)
        e0 = jnp.where(lax.iota(jnp.int32, 16) == 0, 1.0, 0.0)

        @pl.loop(0, BATCH)
        def _(r):
            @pl.loop(0, D_HALF, step=16)
            def _(k):
                rows_v[r, pl.ds(k, 16)] = zero16
            if want_deg:
                ones_v[r, :] = e0
                zdeg_v[r, :] = zero16

        @pl.loop(0, 5)
        def _(j):
            pltpu.sync_copy(rows_v.at[pl.ds(0, 125)],
                            acc_sh.at[pl.ds(rbase + j * 125, 125)])

        if want_deg:
            @pl.when(c == 0)
            def _():
                @pl.loop(0, 5)
                def _(j):
                    pltpu.sync_copy(zdeg_v.at[pl.ds(0, 125)],
                                    deg_sh.at[pl.ds(rbase + j * 125, 125)])

        plsc.subcore_barrier()

        def do_batch(off, n):
            idx_sl = idx_v.at[pl.ds(off, n)]
            dst_sl = dst_v.at[pl.ds(off, n)]
            rows_sl = rows_v.at[pl.ds(0, n)]

            @pl.when(c == 0)
            def _():
                pltpu.sync_copy(x0_hbm.at[idx_sl], rows_sl)

            @pl.when(c == 1)
            def _():
                pltpu.sync_copy(x1_hbm.at[idx_sl], rows_sl)

            pltpu.sync_copy(rows_sl, acc_sh.at[dst_sl], add=True)

            if want_deg:
                @pl.when(c == 0)
                def _():
                    pltpu.sync_copy(ones_v.at[pl.ds(0, n)],
                                    deg_sh.at[dst_sl], add=True)

        @pl.loop(0, NFULL)
        def _(b):
            do_batch(b * BATCH, BATCH)

        do_batch(NFULL * BATCH, TAIL)

        plsc.subcore_barrier()

        @pl.when(c == 0)
        def _():
            pltpu.sync_copy(acc_sh.at[pl.ds(rbase, ROWS_PER)],
                            s_hbm.at[0, pl.ds(rbase, ROWS_PER)])
            if want_deg:
                pltpu.sync_copy(deg_sh.at[pl.ds(rbase, ROWS_PER)],
                                deg_hbm.at[pl.ds(rbase, ROWS_PER)])

        @pl.when(c == 1)
        def _():
            pltpu.sync_copy(acc_sh.at[pl.ds(rbase, ROWS_PER)],
                            s_hbm.at[1, pl.ds(rbase, ROWS_PER)])

    s_type = jax.ShapeDtypeStruct((2, N_NODES, D_HALF), jnp.float32)
    deg_type = jax.ShapeDtypeStruct((N_NODES, 16), jnp.float32)
    scratch = [
        pltpu.VMEM((E_PER,), jnp.int32),
        pltpu.VMEM((E_PER,), jnp.int32),
        pltpu.VMEM((BATCH, D_HALF), jnp.float32),
    ]
    if want_deg:
        scratch += [
            pltpu.VMEM((BATCH, 16), jnp.float32),
            pltpu.VMEM((BATCH, 16), jnp.float32),
        ]
    scratch += [pltpu.VMEM_SHARED((N_NODES, D_HALF), jnp.float32)]
    if want_deg:
        scratch += [pltpu.VMEM_SHARED((N_NODES, 16), jnp.float32)]

    return pl.kernel(
        body,
        out_type=(s_type, deg_type) if want_deg else s_type,
        mesh=plsc.VectorSubcoreMesh(core_axis_name="c", subcore_axis_name="s"),
        compiler_params=pltpu.CompilerParams(use_tc_tiling_on_sc=False),
        scratch_types=scratch,
    )


_segsum_deg = _make_segsum(True)
_segsum_nodeg = _make_segsum(False)


_R = 2000


def _mm0_body(x_ref, w_ref, y0_ref, y1_ref):
    y = jnp.dot(x_ref[...], w_ref[...], preferred_element_type=jnp.float32)
    y0_ref[...] = y[:, :D_HALF]
    y1_ref[...] = y[:, D_HALF:]


def _mm0(x, w0t):
    return pl.pallas_call(
        _mm0_body,
        grid=(N_NODES // _R,),
        in_specs=[
            pl.BlockSpec((_R, D_FEAT), lambda i: (i, 0)),
            pl.BlockSpec((D_FEAT, D_FEAT), lambda i: (0, 0)),
        ],
        out_specs=[
            pl.BlockSpec((_R, D_HALF), lambda i: (i, 0)),
            pl.BlockSpec((_R, D_HALF), lambda i: (i, 0)),
        ],
        out_shape=[jax.ShapeDtypeStruct((N_NODES, D_HALF), jnp.float32)] * 2,
    )(x, w0t)


def _mm1_body(s_ref, deg_ref, w_ref, y0_ref, y1_ref):
    dinv = 1.0 / jnp.maximum(deg_ref[:, 0:1], 1.0)
    h0 = jnp.maximum(s_ref[0], 0.0) * dinv
    h1 = jnp.maximum(s_ref[1], 0.0) * dinv
    y = (jnp.dot(h0, w_ref[:D_HALF, :], preferred_element_type=jnp.float32)
         + jnp.dot(h1, w_ref[D_HALF:, :], preferred_element_type=jnp.float32))
    y0_ref[...] = y[:, :D_HALF]
    y1_ref[...] = y[:, D_HALF:]


def _mm1(s0, deg, w1t):
    return pl.pallas_call(
        _mm1_body,
        grid=(N_NODES // _R,),
        in_specs=[
            pl.BlockSpec((2, _R, D_HALF), lambda i: (0, i, 0)),
            pl.BlockSpec((_R, 16), lambda i: (i, 0)),
            pl.BlockSpec((D_FEAT, D_FEAT), lambda i: (0, 0)),
        ],
        out_specs=[
            pl.BlockSpec((_R, D_HALF), lambda i: (i, 0)),
            pl.BlockSpec((_R, D_HALF), lambda i: (i, 0)),
        ],
        out_shape=[jax.ShapeDtypeStruct((N_NODES, D_HALF), jnp.float32)] * 2,
    )(s0, deg, w1t)


def _scale_body(s_ref, deg_ref, o_ref):
    dinv = 1.0 / jnp.maximum(deg_ref[:, 0:1], 1.0)
    o_ref[:, :D_HALF] = s_ref[0] * dinv
    o_ref[:, D_HALF:] = s_ref[1] * dinv


def _scale(s1, deg):
    return pl.pallas_call(
        _scale_body,
        grid=(N_NODES // _R,),
        in_specs=[
            pl.BlockSpec((2, _R, D_HALF), lambda i: (0, i, 0)),
            pl.BlockSpec((_R, 16), lambda i: (i, 0)),
        ],
        out_specs=pl.BlockSpec((_R, D_FEAT), lambda i: (i, 0)),
        out_shape=jax.ShapeDtypeStruct((N_NODES, D_FEAT), jnp.float32),
    )(s1, deg)


def kernel(x, edge_index, W0, W1):
    src = edge_index[0].astype(jnp.int32)
    dst = edge_index[1].astype(jnp.int32)
    w0t = W0.T
    w1t = W1.T

    y00, y01 = _mm0(x, w0t)
    s0, deg = _segsum_deg(y00, y01, src, dst)
    y10, y11 = _mm1(s0, deg, w1t)
    s1 = _segsum_nodeg(y10, y11, src, dst)
    return _scale(s1, deg)


# speedup vs baseline: 1.9176x; 1.0017x over previous
import jax
import jax.numpy as jnp
from jax import lax
from jax.experimental import pallas as pl
from jax.experimental.pallas import tpu as pltpu
from jax.experimental.pallas import tpu_sc as plsc

N_NODES = 10000
N_EDGES = 160000
D_FEAT = 256
D_HALF = 128

NS = 16
E_PER = N_EDGES // NS
ROWS_PER = N_NODES // NS
BATCH = 128
NFULL = E_PER // BATCH
TAIL = E_PER - NFULL * BATCH


def _make_segsum(want_deg):
    def body(*refs):
        if want_deg:
            (x0_hbm, x1_hbm, src_hbm, dst_hbm, s_hbm, deg_hbm,
             idx_v, dst_v, rows_v, ones_v, zdeg_v, acc_sh, deg_sh) = refs
        else:
            (x0_hbm, x1_hbm, src_hbm, dst_hbm, s_hbm,
             idx_v, dst_v, rows_v, acc_sh) = refs

        c = lax.axis_index("c")
        s = lax.axis_index("s")
        ebase = s * E_PER
        rbase = s * ROWS_PER

        pltpu.sync_copy(src_hbm.at[pl.ds(ebase, E_PER)], idx_v)
        pltpu.sync_copy(dst_hbm.at[pl.ds(ebase, E_PER)], dst_v)

        zero16 = jnp.zeros((16,), jnp.float32)
        e0 = jnp.where(lax.iota(jnp.int32, 16) == 0, 1.0, 0.0)

        @pl.loop(0, BATCH)
        def _(r):
            @pl.loop(0, D_HALF, step=16)
            def _(k):
                rows_v[r, pl.ds(k, 16)] = zero16
            if want_deg:
                ones_v[r, :] = e0
                zdeg_v[r, :] = zero16

        @pl.loop(0, 5)
        def _(j):
            pltpu.sync_copy(rows_v.at[pl.ds(0, 125)],
                            acc_sh.at[pl.ds(rbase + j * 125, 125)])

        if want_deg:
            @pl.when(c == 0)
            def _():
                @pl.loop(0, 5)
                def _(j):
                    pltpu.sync_copy(zdeg_v.at[pl.ds(0, 125)],
                                    deg_sh.at[pl.ds(rbase + j * 125, 125)])

        plsc.subcore_barrier()

        def do_batch(off, n):
            idx_sl = idx_v.at[pl.ds(off, n)]
            dst_sl = dst_v.at[pl.ds(off, n)]
            rows_sl = rows_v.at[pl.ds(0, n)]

            if want_deg:
                @pl.when(c == 0)
                def _():
                    pltpu.sync_copy(ones_v.at[pl.ds(0, n)],
                                    deg_sh.at[dst_sl], add=True)

            @pl.when(c == 0)
            def _():
                pltpu.sync_copy(x0_hbm.at[idx_sl], rows_sl)

            @pl.when(c == 1)
            def _():
                pltpu.sync_copy(x1_hbm.at[idx_sl], rows_sl)

            pltpu.sync_copy(rows_sl, acc_sh.at[dst_sl], add=True)

        @pl.loop(0, NFULL)
        def _(b):
            do_batch(b * BATCH, BATCH)

        do_batch(NFULL * BATCH, TAIL)

        plsc.subcore_barrier()

        @pl.when(c == 0)
        def _():
            pltpu.sync_copy(acc_sh.at[pl.ds(rbase, ROWS_PER)],
                            s_hbm.at[0, pl.ds(rbase, ROWS_PER)])
            if want_deg:
                pltpu.sync_copy(deg_sh.at[pl.ds(rbase, ROWS_PER)],
                                deg_hbm.at[pl.ds(rbase, ROWS_PER)])

        @pl.when(c == 1)
        def _():
            pltpu.sync_copy(acc_sh.at[pl.ds(rbase, ROWS_PER)],
                            s_hbm.at[1, pl.ds(rbase, ROWS_PER)])

    s_type = jax.ShapeDtypeStruct((2, N_NODES, D_HALF), jnp.float32)
    deg_type = jax.ShapeDtypeStruct((N_NODES, 16), jnp.float32)
    scratch = [
        pltpu.VMEM((E_PER,), jnp.int32),
        pltpu.VMEM((E_PER,), jnp.int32),
        pltpu.VMEM((BATCH, D_HALF), jnp.float32),
    ]
    if want_deg:
        scratch += [
            pltpu.VMEM((BATCH, 16), jnp.float32),
            pltpu.VMEM((BATCH, 16), jnp.float32),
        ]
    scratch += [pltpu.VMEM_SHARED((N_NODES, D_HALF), jnp.float32)]
    if want_deg:
        scratch += [pltpu.VMEM_SHARED((N_NODES, 16), jnp.float32)]

    return pl.kernel(
        body,
        out_type=(s_type, deg_type) if want_deg else s_type,
        mesh=plsc.VectorSubcoreMesh(core_axis_name="c", subcore_axis_name="s"),
        compiler_params=pltpu.CompilerParams(use_tc_tiling_on_sc=False),
        scratch_types=scratch,
    )


_segsum_deg = _make_segsum(True)
_segsum_nodeg = _make_segsum(False)


_R = 2000


def _mm0_body(x_ref, w_ref, y0_ref, y1_ref):
    y = jnp.dot(x_ref[...], w_ref[...], preferred_element_type=jnp.float32)
    y0_ref[...] = y[:, :D_HALF]
    y1_ref[...] = y[:, D_HALF:]


def _mm0(x, w0t):
    return pl.pallas_call(
        _mm0_body,
        grid=(N_NODES // _R,),
        in_specs=[
            pl.BlockSpec((_R, D_FEAT), lambda i: (i, 0)),
            pl.BlockSpec((D_FEAT, D_FEAT), lambda i: (0, 0)),
        ],
        out_specs=[
            pl.BlockSpec((_R, D_HALF), lambda i: (i, 0)),
            pl.BlockSpec((_R, D_HALF), lambda i: (i, 0)),
        ],
        out_shape=[jax.ShapeDtypeStruct((N_NODES, D_HALF), jnp.float32)] * 2,
    )(x, w0t)


def _mm1_body(s_ref, deg_ref, w_ref, y0_ref, y1_ref):
    dinv = 1.0 / jnp.maximum(deg_ref[:, 0:1], 1.0)
    h0 = jnp.maximum(s_ref[0], 0.0) * dinv
    h1 = jnp.maximum(s_ref[1], 0.0) * dinv
    y = (jnp.dot(h0, w_ref[:D_HALF, :], preferred_element_type=jnp.float32)
         + jnp.dot(h1, w_ref[D_HALF:, :], preferred_element_type=jnp.float32))
    y0_ref[...] = y[:, :D_HALF]
    y1_ref[...] = y[:, D_HALF:]


def _mm1(s0, deg, w1t):
    return pl.pallas_call(
        _mm1_body,
        grid=(N_NODES // _R,),
        in_specs=[
            pl.BlockSpec((2, _R, D_HALF), lambda i: (0, i, 0)),
            pl.BlockSpec((_R, 16), lambda i: (i, 0)),
            pl.BlockSpec((D_FEAT, D_FEAT), lambda i: (0, 0)),
        ],
        out_specs=[
            pl.BlockSpec((_R, D_HALF), lambda i: (i, 0)),
            pl.BlockSpec((_R, D_HALF), lambda i: (i, 0)),
        ],
        out_shape=[jax.ShapeDtypeStruct((N_NODES, D_HALF), jnp.float32)] * 2,
    )(s0, deg, w1t)


def _scale_body(s_ref, deg_ref, o_ref):
    dinv = 1.0 / jnp.maximum(deg_ref[:, 0:1], 1.0)
    o_ref[:, :D_HALF] = s_ref[0] * dinv
    o_ref[:, D_HALF:] = s_ref[1] * dinv


def _scale(s1, deg):
    return pl.pallas_call(
        _scale_body,
        grid=(N_NODES // _R,),
        in_specs=[
            pl.BlockSpec((2, _R, D_HALF), lambda i: (0, i, 0)),
            pl.BlockSpec((_R, 16), lambda i: (i, 0)),
        ],
        out_specs=pl.BlockSpec((_R, D_FEAT), lambda i: (i, 0)),
        out_shape=jax.ShapeDtypeStruct((N_NODES, D_FEAT), jnp.float32),
    )(s1, deg)


def kernel(x, edge_index, W0, W1):
    src = edge_index[0].astype(jnp.int32)
    dst = edge_index[1].astype(jnp.int32)
    w0t = W0.T
    w1t = W1.T

    y00, y01 = _mm0(x, w0t)
    s0, deg = _segsum_deg(y00, y01, src, dst)
    y10, y11 = _mm1(s0, deg, w1t)
    s1 = _segsum_nodeg(y10, y11, src, dst)
    return _scale(s1, deg)
